# baseline (speedup 1.0000x reference)
import jax
import jax.numpy as jnp
from jax.experimental import pallas as pl
from jax.experimental.pallas import tpu as pltpu

B = 64
C = 32
T = 128
FQ = 128
TFQ = T * FQ
K = int(T * FQ * 0.01)
KPAD = 256


def _shift_t(a, s):
    z = jnp.zeros((C, 1, FQ), a.dtype)
    if s == 1:
        return jnp.concatenate([a[:, 1:, :], z], axis=1)
    if s == -1:
        return jnp.concatenate([z, a[:, :-1, :]], axis=1)
    return a


def _shift_f(a, s):
    z = jnp.zeros((C, T, 1), a.dtype)
    if s == 1:
        return jnp.concatenate([a[:, :, 1:], z], axis=2)
    if s == -1:
        return jnp.concatenate([z, a[:, :, :-1]], axis=2)
    return a


def _denom(e, scr):
    eb = e.reshape(16, 8, 128)
    eb = jnp.transpose(eb, (2, 1, 0)).reshape(1024, 16)
    scr[...] = eb

    def outer(c8, acc):
        chunk = scr[pl.ds(c8 * 8, 8), :]
        for k in range(8):
            acc = acc + chunk[k:k + 1, :]
        return acc

    acc = jax.lax.fori_loop(0, 128, outer, jnp.zeros((1, 16), jnp.float32))
    d = acc[:, 0:1]
    for i in range(1, 16):
        d = d + acc[:, i:i + 1]
    return d


def _body(x_ref, wf_ref, sw_ref, vw_ref, cb_ref, sb_ref, vb_ref,
          s_ref, idx_ref, lat_ref, dscr_ref, tscr_ref):
    x3 = x_ref[0]
    xb = x3.astype(jnp.bfloat16)
    cols = [_shift_f(_shift_t(xb, st), sf).reshape(C, TFQ)
            for st in (-1, 0, 1) for sf in (-1, 0, 1)]
    xcol = jnp.concatenate(cols, axis=0)
    conv = jax.lax.dot_general(wf_ref[...], xcol, (((1,), (0,)), ((), ())),
                               preferred_element_type=jnp.float32)
    x1 = conv + cb_ref[...]

    n = jnp.sqrt(jnp.sum(x1 * x1, axis=0, keepdims=True))
    normed = x1 / (n + 1e-08)
    sw = sw_ref[...]
    lg = jax.lax.dot_general(sw.T.astype(jnp.bfloat16),
                             normed.astype(jnp.bfloat16),
                             (((1,), (0,)), ((), ())),
                             preferred_element_type=jnp.float32)
    lg = lg + sb_ref[...]

    m = jnp.max(lg)
    e = jnp.exp(lg - m)
    s = e / _denom(e, dscr_ref)
    s_ref[0] = s.reshape(T, FQ)

    tscr_ref[...] = s.reshape(T, FQ)
    gm = jnp.max(s.reshape(16, 8, 128), axis=(1, 2)).reshape(1, 16)
    giota = jax.lax.broadcasted_iota(jnp.int32, (1, 16), 1)
    liota = (jax.lax.broadcasted_iota(jnp.int32, (8, 128), 0) * 128
             + jax.lax.broadcasted_iota(jnp.int32, (8, 128), 1))
    kiota = jax.lax.broadcasted_iota(jnp.int32, (1, KPAD), 1)

    def step(k, carry):
        gm, idxacc = carry
        mv = jnp.max(gm)
        g = jnp.min(jnp.where(gm == mv, giota, 16))
        blk = tscr_ref[pl.ds(g * 8, 8), :]
        loc = jnp.min(jnp.where(blk == mv, liota, 1024))
        idxacc = jnp.where(kiota == k, g * 1024 + loc, idxacc)
        blk = jnp.where(liota == loc, -1.0, blk)
        tscr_ref[pl.ds(g * 8, 8), :] = blk
        gm = jnp.where(giota == g, jnp.max(blk), gm)
        return gm, idxacc

    _, idxacc = jax.lax.fori_loop(
        0, K, step, (gm, jnp.zeros((1, KPAD), jnp.int32)))
    idxsl = idxacc[:, :K]
    idx_ref[0] = idxsl

    oht = (jax.lax.broadcasted_iota(jnp.int32, (TFQ, K), 0)
           == idxsl).astype(jnp.bfloat16)
    latg = jax.lax.dot_general(x1.astype(jnp.bfloat16), oht,
                               (((1,), (0,)), ((), ())),
                               preferred_element_type=jnp.float32)
    latt = jax.lax.dot_general(vw_ref[...], latg.astype(jnp.bfloat16),
                               (((1,), (0,)), ((), ())),
                               preferred_element_type=jnp.float32)
    latt = latt + vb_ref[...]
    lat_ref[0] = latt.T


def kernel(x, conv_w, conv_b, values_w, values_b, select_w, select_b):
    wf = jnp.transpose(conv_w, (0, 2, 3, 1)).reshape(C, 9 * C)
    wf = wf.astype(jnp.bfloat16)
    sw = select_w.reshape(1, C).T
    vw = values_w.reshape(C, C).astype(jnp.bfloat16)
    cb = conv_b.reshape(C, 1)
    sb = select_b.reshape(1, 1)
    vb = values_b.reshape(C, 1)

    s_out, idx_out, lat_out = pl.pallas_call(
        _body,
        grid=(B,),
        in_specs=[
            pl.BlockSpec((1, C, T, FQ), lambda i: (i, 0, 0, 0)),
            pl.BlockSpec((C, 9 * C), lambda i: (0, 0)),
            pl.BlockSpec((C, 1), lambda i: (0, 0)),
            pl.BlockSpec((C, C), lambda i: (0, 0)),
            pl.BlockSpec((C, 1), lambda i: (0, 0)),
            pl.BlockSpec((1, 1), lambda i: (0, 0)),
            pl.BlockSpec((C, 1), lambda i: (0, 0)),
        ],
        out_specs=[
            pl.BlockSpec((1, T, FQ), lambda i: (i, 0, 0)),
            pl.BlockSpec((1, 1, K), lambda i: (i, 0, 0)),
            pl.BlockSpec((1, K, C), lambda i: (i, 0, 0)),
        ],
        out_shape=[
            jax.ShapeDtypeStruct((B, T, FQ), jnp.float32),
            jax.ShapeDtypeStruct((B, 1, K), jnp.int32),
            jax.ShapeDtypeStruct((B, K, C), jnp.float32),
        ],
        scratch_shapes=[pltpu.VMEM((1024, 16), jnp.float32),
                        pltpu.VMEM((T, FQ), jnp.float32)],
        compiler_params=pltpu.CompilerParams(
            dimension_semantics=("parallel",)),
    )(x, wf, sw, vw, cb, sb, vb)

    indices = idx_out.reshape(B, K)
    latents = lat_out.reshape(B * K, C)
    return (indices, latents, s_out)


# speedup vs baseline: 1.7704x; 1.7704x over previous
import jax
import jax.numpy as jnp
from jax.experimental import pallas as pl
from jax.experimental.pallas import tpu as pltpu

B = 64
C = 32
T = 128
FQ = 128
TFQ = T * FQ
K = int(T * FQ * 0.01)
KPAD = 256


def _shift_t(a, s):
    z = jnp.zeros((C, 1, FQ), a.dtype)
    if s == 1:
        return jnp.concatenate([a[:, 1:, :], z], axis=1)
    if s == -1:
        return jnp.concatenate([z, a[:, :-1, :]], axis=1)
    return a


def _shift_f(a, s):
    z = jnp.zeros((C, T, 1), a.dtype)
    if s == 1:
        return jnp.concatenate([a[:, :, 1:], z], axis=2)
    if s == -1:
        return jnp.concatenate([z, a[:, :, :-1]], axis=2)
    return a


def _denom(e, scr):
    eb = e.reshape(16, 8, 128)
    eb = jnp.transpose(eb, (2, 1, 0)).reshape(1024, 16)
    scr[...] = eb

    def outer(c64, acc):
        chunk = scr[pl.ds(c64 * 64, 64), :]
        for k in range(64):
            acc = acc + chunk[k:k + 1, :]
        return acc

    acc = jax.lax.fori_loop(0, 16, outer, jnp.zeros((1, 16), jnp.float32))
    d = acc[:, 0:1]
    for i in range(1, 16):
        d = d + acc[:, i:i + 1]
    return d


def _body(x_ref, wf_ref, sw_ref, vw_ref, cb_ref, sb_ref, vb_ref,
          s_ref, idx_ref, lat_ref, dscr_ref):
    x3 = x_ref[0]
    xb = x3.astype(jnp.bfloat16)
    cols = [_shift_f(_shift_t(xb, st), sf).reshape(C, TFQ)
            for st in (-1, 0, 1) for sf in (-1, 0, 1)]
    xcol = jnp.concatenate(cols, axis=0)
    conv = jax.lax.dot_general(wf_ref[...], xcol, (((1,), (0,)), ((), ())),
                               preferred_element_type=jnp.float32)
    x1 = conv + cb_ref[...]

    n = jnp.sqrt(jnp.sum(x1 * x1, axis=0, keepdims=True))
    normed = x1 / (n + 1e-08)
    sw = sw_ref[...]
    lg = jax.lax.dot_general(sw.T.astype(jnp.bfloat16),
                             normed.astype(jnp.bfloat16),
                             (((1,), (0,)), ((), ())),
                             preferred_element_type=jnp.float32)
    lg = lg + sb_ref[...]

    m = jnp.max(lg)
    e = jnp.exp(lg - m)
    s = e / _denom(e, dscr_ref)
    s_ref[0] = s.reshape(T, FQ)

    sv0 = s.reshape(8, TFQ // 8)
    iota = (jax.lax.broadcasted_iota(jnp.int32, (8, TFQ // 8), 0) * (TFQ // 8)
            + jax.lax.broadcasted_iota(jnp.int32, (8, TFQ // 8), 1))
    kiota = jax.lax.broadcasted_iota(jnp.int32, (1, KPAD), 1)

    def step(k, carry):
        sv, idxacc, iot, kio = carry
        mv = jnp.max(sv)
        cand = jnp.where(sv == mv, iot, TFQ)
        pick = jnp.min(cand)
        idxacc = jnp.where(kio == k, pick, idxacc)
        sv = jnp.where(iot == pick, -1.0, sv)
        return sv, idxacc, iot, kio

    _, idxacc, _, _ = jax.lax.fori_loop(
        0, K, step, (sv0, jnp.zeros((1, KPAD), jnp.int32), iota, kiota))
    idxsl = idxacc[:, :K]
    idx_ref[0] = idxsl

    oht = (jax.lax.broadcasted_iota(jnp.int32, (TFQ, K), 0)
           == idxsl).astype(jnp.bfloat16)
    latg = jax.lax.dot_general(x1.astype(jnp.bfloat16), oht,
                               (((1,), (0,)), ((), ())),
                               preferred_element_type=jnp.float32)
    latt = jax.lax.dot_general(vw_ref[...], latg.astype(jnp.bfloat16),
                               (((1,), (0,)), ((), ())),
                               preferred_element_type=jnp.float32)
    latt = latt + vb_ref[...]
    lat_ref[0] = latt.T


def kernel(x, conv_w, conv_b, values_w, values_b, select_w, select_b):
    wf = jnp.transpose(conv_w, (0, 2, 3, 1)).reshape(C, 9 * C)
    wf = wf.astype(jnp.bfloat16)
    sw = select_w.reshape(1, C).T
    vw = values_w.reshape(C, C).astype(jnp.bfloat16)
    cb = conv_b.reshape(C, 1)
    sb = select_b.reshape(1, 1)
    vb = values_b.reshape(C, 1)

    s_out, idx_out, lat_out = pl.pallas_call(
        _body,
        grid=(B,),
        in_specs=[
            pl.BlockSpec((1, C, T, FQ), lambda i: (i, 0, 0, 0)),
            pl.BlockSpec((C, 9 * C), lambda i: (0, 0)),
            pl.BlockSpec((C, 1), lambda i: (0, 0)),
            pl.BlockSpec((C, C), lambda i: (0, 0)),
            pl.BlockSpec((C, 1), lambda i: (0, 0)),
            pl.BlockSpec((1, 1), lambda i: (0, 0)),
            pl.BlockSpec((C, 1), lambda i: (0, 0)),
        ],
        out_specs=[
            pl.BlockSpec((1, T, FQ), lambda i: (i, 0, 0)),
            pl.BlockSpec((1, 1, K), lambda i: (i, 0, 0)),
            pl.BlockSpec((1, K, C), lambda i: (i, 0, 0)),
        ],
        out_shape=[
            jax.ShapeDtypeStruct((B, T, FQ), jnp.float32),
            jax.ShapeDtypeStruct((B, 1, K), jnp.int32),
            jax.ShapeDtypeStruct((B, K, C), jnp.float32),
        ],
        scratch_shapes=[pltpu.VMEM((1024, 16), jnp.float32)],
        compiler_params=pltpu.CompilerParams(
            dimension_semantics=("parallel",)),
    )(x, wf, sw, vw, cb, sb, vb)

    indices = idx_out.reshape(B, K)
    latents = lat_out.reshape(B * K, C)
    return (indices, latents, s_out)
